# baseline (speedup 1.0000x reference)
import functools

import jax
import jax.numpy as jnp
from jax import lax
from jax.experimental import pallas as pl
from jax.experimental.pallas import tpu as pltpu
from jax.experimental.pallas import tpu_sc as plsc

N = 10000
E = 320000
D_IN = 128
D_EDGE = 4
H = 64
HID = 32
G = 64
L = 3

NC = 2
NS = 16
NW = NC * NS

EH = E // 2
CH = 40
EPW = EH // NW
NCH = EPW // CH
EPC = EH // NC
EPT = EPC // NS
RPT = N // NS
NB = 5

_SC_MESH = plsc.VectorSubcoreMesh(core_axis_name="c", subcore_axis_name="s")
_SC_PARAMS = pltpu.CompilerParams(use_tc_tiling_on_sc=False)


@functools.partial(
    pl.kernel,
    mesh=_SC_MESH,
    out_type=jax.ShapeDtypeStruct((EH // 4, 128), jnp.float32),
    scratch_types=[
        pltpu.VMEM((NCH, CH), jnp.int32),
        pltpu.VMEM((NCH, CH), jnp.int32),
        pltpu.VMEM((RPT, HID), jnp.float32),
        pltpu.VMEM_SHARED((N, HID), jnp.float32),
        pltpu.VMEM_SHARED((N, HID), jnp.float32),
        [pltpu.VMEM((CH, HID), jnp.float32)] * NB,
        [pltpu.VMEM((CH, HID), jnp.float32)] * NB,
        [pltpu.VMEM((CH // 4, 128), jnp.float32)] * NB,
        [pltpu.SemaphoreType.DMA] * NB,
        [pltpu.SemaphoreType.DMA] * NB,
        [pltpu.SemaphoreType.DMA] * NB,
    ],
    compiler_params=_SC_PARAMS,
)
def _sc_gather(tab_a, tab_b, dst_r, src_r, out, idx_d, idx_s, stage, sh_a,
               sh_b, buf_a, buf_b, buf_o, sem_a, sem_b, sem_o):
    cid = lax.axis_index("c")
    sid = lax.axis_index("s")
    wid = sid * NC + cid

    pltpu.sync_copy(dst_r.at[wid], idx_d)
    pltpu.sync_copy(src_r.at[wid], idx_s)
    r0 = sid * RPT
    pltpu.sync_copy(tab_a.at[pl.ds(r0, RPT)], stage)
    pltpu.sync_copy(stage, sh_a.at[pl.ds(r0, RPT)])
    pltpu.sync_copy(tab_b.at[pl.ds(r0, RPT)], stage)
    pltpu.sync_copy(stage, sh_b.at[pl.ds(r0, RPT)])
    plsc.subcore_barrier()

    def issue(b, i):
        pltpu.async_copy(sh_a.at[idx_d.at[i]], buf_a[b], sem_a[b])
        pltpu.async_copy(sh_b.at[idx_s.at[i]], buf_b[b], sem_b[b])

    for b in range(NB):
        issue(b, b)

    def outer(g, carry):
        for b in range(NB):
            i = g * NB + b
            pltpu.make_async_copy(tab_a.at[pl.ds(0, CH)], buf_a[b],
                                  sem_a[b]).wait()
            pltpu.make_async_copy(tab_a.at[pl.ds(0, CH)], buf_b[b],
                                  sem_b[b]).wait()

            @pl.when(g > 0)
            def _():
                pltpu.make_async_copy(buf_o[b], out.at[pl.ds(0, CH // 4)],
                                      sem_o[b]).wait()

            for j in range(CH * HID // 16):
                sl_a = (j // (HID // 16), pl.ds((j % (HID // 16)) * 16, 16))
                sl_o = (j // 8, pl.ds((j % 8) * 16, 16))
                buf_o[b][sl_o] = buf_a[b][sl_a] + buf_b[b][sl_a]
            pltpu.async_copy(
                buf_o[b],
                out.at[pl.ds((wid * EPW + i * CH) // 4, CH // 4)],
                sem_o[b])

            @pl.when(i + NB < NCH)
            def _():
                issue(b, i + NB)
        return carry

    lax.fori_loop(0, NCH // NB, outer, 0)
    for b in range(NB):
        pltpu.make_async_copy(buf_o[b], out.at[pl.ds(0, CH // 4)],
                              sem_o[b]).wait()


@functools.partial(
    pl.kernel,
    mesh=_SC_MESH,
    out_type=jax.ShapeDtypeStruct((NC, N, H), jnp.float32),
    scratch_types=[
        pltpu.VMEM((NCH, CH), jnp.int32),
        pltpu.VMEM((RPT // 5, H), jnp.float32),
        pltpu.VMEM_SHARED((N, H), jnp.float32),
        [pltpu.VMEM((CH, H), jnp.float32)] * NB,
        [pltpu.SemaphoreType.DMA] * NB,
    ],
    compiler_params=_SC_PARAMS,
)
def _sc_scatter(m_h, dst_r, out, idx, stage, acc, buf_m, sem_m):
    cid = lax.axis_index("c")
    sid = lax.axis_index("s")
    r0 = sid * RPT
    e0 = cid * EPC + sid * EPT

    pltpu.sync_copy(dst_r.at[cid, sid], idx)

    def zrow(r, carry):
        for c in range(H // 16):
            stage[r, pl.ds(c * 16, 16)] = jnp.zeros((16,), jnp.float32)
        return carry

    lax.fori_loop(0, RPT // 5, zrow, 0)
    for k in range(5):
        pltpu.sync_copy(stage, acc.at[pl.ds(r0 + k * (RPT // 5), RPT // 5)])
    plsc.subcore_barrier()

    def issue(b, i):
        pltpu.async_copy(m_h.at[pl.ds(e0 + i * CH, CH)], buf_m[b], sem_m[b])

    for b in range(NB):
        issue(b, b)

    def outer(g, carry):
        for b in range(NB):
            i = g * NB + b
            pltpu.make_async_copy(m_h.at[pl.ds(0, CH)], buf_m[b],
                                  sem_m[b]).wait()
            pltpu.sync_copy(buf_m[b], acc.at[idx.at[i]], add=True)

            @pl.when(i + NB < NCH)
            def _():
                issue(b, i + NB)
        return carry

    lax.fori_loop(0, NCH // NB, outer, 0)
    plsc.subcore_barrier()
    for k in range(5):
        rk = r0 + k * (RPT // 5)
        pltpu.sync_copy(acc.at[pl.ds(rk, RPT // 5)], stage)
        pltpu.sync_copy(stage, out.at[cid, pl.ds(rk, RPT // 5)])


def _tc_init_body(x, w_in, b_in, wa, wb, h_out, a_out, b_out):
    h = jnp.dot(x[:], w_in[:], preferred_element_type=jnp.float32) + b_in[:]
    h_out[:] = h
    a_out[:] = jnp.dot(h, wa[:], preferred_element_type=jnp.float32)
    b_out[:] = jnp.dot(h, wb[:], preferred_element_type=jnp.float32)


_tc_init = pl.pallas_call(
    _tc_init_body,
    out_shape=[
        jax.ShapeDtypeStruct((N, H), jnp.float32),
        jax.ShapeDtypeStruct((N, HID), jnp.float32),
        jax.ShapeDtypeStruct((N, HID), jnp.float32),
    ],
)

BR = 1600


def _tc_msg_body(zg4, ea16, w1cp, b1p, w2d, b2d, s2d, be2d, m_out):
    ea = ea16[:]
    rid = lax.broadcasted_iota(jnp.int32, (BR, 128), 0) % 8
    ce = jnp.zeros((BR, 128), jnp.float32)
    for p in range(8):
        t = jnp.dot(ea[:, 16 * p:16 * p + 16], w1cp[:],
                    preferred_element_type=jnp.float32)
        tr = jnp.broadcast_to(t[:, None, :], (BR // 8, 8, 128))
        ce = jnp.where(rid == p, tr.reshape(BR, 128), ce)
    u = jnp.maximum(zg4[:] + ce + b1p[:], 0.0)
    even = jnp.maximum(
        jnp.dot(u[:, :64], w2d[:], preferred_element_type=jnp.float32)
        + b2d[:], 0.0) * s2d[:] + be2d[:]
    odd = jnp.maximum(
        jnp.dot(u[:, 64:], w2d[:], preferred_element_type=jnp.float32)
        + b2d[:], 0.0) * s2d[:] + be2d[:]
    m_out[:] = jnp.stack([even, odd], axis=1).reshape(2 * BR, 128)


def _make_tc_msg(off):
    return pl.pallas_call(
        _tc_msg_body,
        grid=(EH // (4 * BR),),
        in_specs=[
            pl.BlockSpec((BR, 128), lambda i: (i, 0)),
            pl.BlockSpec((BR // 8, 128), lambda i: (i + off, 0)),
            pl.BlockSpec((16, 128), lambda i: (0, 0)),
            pl.BlockSpec((1, 128), lambda i: (0, 0)),
            pl.BlockSpec((64, 128), lambda i: (0, 0)),
            pl.BlockSpec((1, 128), lambda i: (0, 0)),
            pl.BlockSpec((1, 128), lambda i: (0, 0)),
            pl.BlockSpec((1, 128), lambda i: (0, 0)),
        ],
        out_specs=pl.BlockSpec((2 * BR, 128), lambda i: (i, 0)),
        out_shape=jax.ShapeDtypeStruct((EH // 2, 128), jnp.float32),
    )


_tc_msg_a = _make_tc_msg(0)
_tc_msg_b = _make_tc_msg(EH // (4 * BR))


def _tc_upd_mid_body(h, pa, pb, u1a, u1b, c1, u2, c2, s2, be2, wa, wb,
                     h_out, a_out, b_out):
    aggr = pa[0] + pa[1] + pb[0] + pb[1]
    v = jnp.maximum(
        jnp.dot(h[:], u1a[:], preferred_element_type=jnp.float32)
        + jnp.dot(aggr, u1b[:], preferred_element_type=jnp.float32)
        + c1[:], 0.0)
    up = jnp.maximum(
        jnp.dot(v, u2[:], preferred_element_type=jnp.float32) + c2[:], 0.0)
    hn = h[:] + up * s2[:] + be2[:]
    h_out[:] = hn
    a_out[:] = jnp.dot(hn, wa[:], preferred_element_type=jnp.float32)
    b_out[:] = jnp.dot(hn, wb[:], preferred_element_type=jnp.float32)


def _tc_upd_last_body(h, pa, pb, u1a, u1b, c1, u2, c2, s2, be2, h_out):
    aggr = pa[0] + pa[1] + pb[0] + pb[1]
    v = jnp.maximum(
        jnp.dot(h[:], u1a[:], preferred_element_type=jnp.float32)
        + jnp.dot(aggr, u1b[:], preferred_element_type=jnp.float32)
        + c1[:], 0.0)
    up = jnp.maximum(
        jnp.dot(v, u2[:], preferred_element_type=jnp.float32) + c2[:], 0.0)
    h_out[:] = h[:] + up * s2[:] + be2[:]


_tc_upd_mid = pl.pallas_call(
    _tc_upd_mid_body,
    out_shape=[
        jax.ShapeDtypeStruct((N, H), jnp.float32),
        jax.ShapeDtypeStruct((N, HID), jnp.float32),
        jax.ShapeDtypeStruct((N, HID), jnp.float32),
    ],
)

_tc_upd_last = pl.pallas_call(
    _tc_upd_last_body,
    out_shape=jax.ShapeDtypeStruct((N, H), jnp.float32),
)


def _tc_final_body(h, batch, w_out, b_out, y_out):
    hv = h[:]
    mu = jnp.mean(hv, axis=-1, keepdims=True)
    var = jnp.mean(hv * hv, axis=-1, keepdims=True) - mu * mu
    hn = (hv - mu) * lax.rsqrt(var + 1e-5)
    gid = lax.broadcasted_iota(jnp.int32, (N, G), 1)
    onehot = jnp.where(batch[:] == gid, 1.0, 0.0).astype(jnp.float32)
    sums = lax.dot_general(onehot, hn, (((0,), (0,)), ((), ())),
                           preferred_element_type=jnp.float32)
    cnt = jnp.sum(onehot, axis=0).reshape(G, 1)
    pooled = sums / jnp.maximum(cnt, 1.0)
    y_out[:] = (jnp.dot(pooled, w_out[:], preferred_element_type=jnp.float32)
                + b_out[:])


_tc_final = pl.pallas_call(
    _tc_final_body,
    out_shape=jax.ShapeDtypeStruct((G, 3), jnp.float32),
)


def _fold_seq(p):
    s = 1.0 / jnp.sqrt(jnp.float32(1.0 + 1e-5))
    g1s = p["g1"] * s
    w1 = p["W1"] * g1s
    b1 = p["b1"] * g1s + p["be1"]
    s2 = p["g2"] * s
    return w1, b1, p["W2"], p["b2"], s2, p["be2"]


def kernel(x, edge_index, edge_attr, batch, params):
    src = edge_index[0]
    dst = edge_index[1]
    halves = []
    for hf in range(2):
        sl = slice(hf * EH, (hf + 1) * EH)
        halves.append(dict(
            src_g=src[sl].reshape(NW, NCH, CH),
            dst_g=dst[sl].reshape(NW, NCH, CH),
            dst_s=dst[sl].reshape(NC, NS, NCH, CH),
        ))
    ea16 = edge_attr.reshape(E // 32, 128)
    batch2d = batch.reshape(N, 1)
    bd = jax.scipy.linalg.block_diag

    folded = []
    for lp in params["layers"]:
        w1, b1, w2, b2, s2, be2 = _fold_seq(lp["msg"])
        uw1, uc1, uu2, uc2, us2, ube2 = _fold_seq(lp["upd"])
        w1c = w1[2 * H:]
        folded.append(dict(
            wa=w1[:H], wb=w1[H:2 * H],
            w1cp=bd(w1c, w1c, w1c, w1c),
            b1p=jnp.tile(b1, 4).reshape(1, 128),
            w2d=bd(w2, w2),
            b2d=jnp.tile(b2, 2).reshape(1, 128),
            s2d=jnp.tile(s2, 2).reshape(1, 128),
            be2d=jnp.tile(be2, 2).reshape(1, 128),
            u1a=uw1[:H], u1b=uw1[H:], c1=uc1.reshape(1, HID),
            u2=uu2, c2=uc2.reshape(1, H), us2=us2.reshape(1, H),
            ube2=ube2.reshape(1, H),
        ))

    h, a_tab, b_tab = _tc_init(x, params["Win"], params["bin"].reshape(1, H),
                               folded[0]["wa"], folded[0]["wb"])

    for li in range(L):
        f = folded[li]
        ha, hb = halves
        zg_a = _sc_gather(a_tab, b_tab, ha["dst_g"], ha["src_g"])
        m_a = _tc_msg_a(zg_a, ea16, f["w1cp"], f["b1p"], f["w2d"], f["b2d"],
                        f["s2d"], f["be2d"])
        zg_b = _sc_gather(a_tab, b_tab, hb["dst_g"], hb["src_g"])
        m_b = _tc_msg_b(zg_b, ea16, f["w1cp"], f["b1p"], f["w2d"], f["b2d"],
                        f["s2d"], f["be2d"])
        p_a = _sc_scatter(m_a.reshape(EH, H), ha["dst_s"])
        p_b = _sc_scatter(m_b.reshape(EH, H), hb["dst_s"])
        if li + 1 < L:
            nf = folded[li + 1]
            h, a_tab, b_tab = _tc_upd_mid(
                h, p_a, p_b, f["u1a"], f["u1b"], f["c1"], f["u2"], f["c2"],
                f["us2"], f["ube2"], nf["wa"], nf["wb"])
        else:
            h = _tc_upd_last(h, p_a, p_b, f["u1a"], f["u1b"], f["c1"],
                             f["u2"], f["c2"], f["us2"], f["ube2"])

    return _tc_final(h, batch2d, params["Wout"], params["bout"].reshape(1, 3))


# speedup vs baseline: 1.2464x; 1.2464x over previous
import functools

import jax
import jax.numpy as jnp
from jax import lax
from jax.experimental import pallas as pl
from jax.experimental.pallas import tpu as pltpu
from jax.experimental.pallas import tpu_sc as plsc

N = 10000
E = 320000
D_IN = 128
D_EDGE = 4
H = 64
HID = 32
G = 64
L = 3

NC = 2
NS = 16
NW = NC * NS

EH = E // 2
CH = 40
EPW = EH // NW
NCH = EPW // CH
EPC = EH // NC
EPT = EPC // NS
RPT = N // NS
NB = 5

_SC_MESH = plsc.VectorSubcoreMesh(core_axis_name="c", subcore_axis_name="s")
_SC_PARAMS = pltpu.CompilerParams(use_tc_tiling_on_sc=False)


@functools.partial(
    pl.kernel,
    mesh=_SC_MESH,
    out_type=jax.ShapeDtypeStruct((EH // 4, 128), jnp.float32),
    scratch_types=[
        pltpu.VMEM((NCH, CH), jnp.int32),
        pltpu.VMEM((NCH, CH), jnp.int32),
        pltpu.VMEM((RPT, HID), jnp.float32),
        pltpu.VMEM_SHARED((N, HID), jnp.float32),
        pltpu.VMEM_SHARED((N, HID), jnp.float32),
        [pltpu.VMEM((CH, HID), jnp.float32)] * NB,
        [pltpu.VMEM((CH, HID), jnp.float32)] * NB,
        [pltpu.VMEM((CH // 4, 128), jnp.float32)] * NB,
        [pltpu.SemaphoreType.DMA] * NB,
        [pltpu.SemaphoreType.DMA] * NB,
        [pltpu.SemaphoreType.DMA] * NB,
    ],
    compiler_params=_SC_PARAMS,
)
def _sc_gather(tab_a, tab_b, dst_r, src_r, out, idx_d, idx_s, stage, sh_a,
               sh_b, buf_a, buf_b, buf_o, sem_a, sem_b, sem_o):
    cid = lax.axis_index("c")
    sid = lax.axis_index("s")
    wid = sid * NC + cid

    pltpu.sync_copy(dst_r.at[wid], idx_d)
    pltpu.sync_copy(src_r.at[wid], idx_s)
    r0 = sid * RPT
    pltpu.sync_copy(tab_a.at[pl.ds(r0, RPT)], stage)
    pltpu.sync_copy(stage, sh_a.at[pl.ds(r0, RPT)])
    pltpu.sync_copy(tab_b.at[pl.ds(r0, RPT)], stage)
    pltpu.sync_copy(stage, sh_b.at[pl.ds(r0, RPT)])
    plsc.subcore_barrier()

    def issue(b, i):
        pltpu.async_copy(sh_a.at[idx_d.at[i]], buf_a[b], sem_a[b])
        pltpu.async_copy(sh_b.at[idx_s.at[i]], buf_b[b], sem_b[b])

    for b in range(NB):
        issue(b, b)

    def outer(g, carry):
        for b in range(NB):
            i = g * NB + b
            pltpu.make_async_copy(tab_a.at[pl.ds(0, CH)], buf_a[b],
                                  sem_a[b]).wait()
            pltpu.make_async_copy(tab_a.at[pl.ds(0, CH)], buf_b[b],
                                  sem_b[b]).wait()

            @pl.when(g > 0)
            def _():
                pltpu.make_async_copy(buf_o[b], out.at[pl.ds(0, CH // 4)],
                                      sem_o[b]).wait()

            for j in range(CH * HID // 16):
                sl_a = (j // (HID // 16), pl.ds((j % (HID // 16)) * 16, 16))
                sl_o = (j // 8, pl.ds((j % 8) * 16, 16))
                buf_o[b][sl_o] = buf_a[b][sl_a] + buf_b[b][sl_a]
            pltpu.async_copy(
                buf_o[b],
                out.at[pl.ds((wid * EPW + i * CH) // 4, CH // 4)],
                sem_o[b])

            @pl.when(i + NB < NCH)
            def _():
                issue(b, i + NB)
        return carry

    lax.fori_loop(0, NCH // NB, outer, 0)
    for b in range(NB):
        pltpu.make_async_copy(buf_o[b], out.at[pl.ds(0, CH // 4)],
                              sem_o[b]).wait()


@functools.partial(
    pl.kernel,
    mesh=_SC_MESH,
    out_type=jax.ShapeDtypeStruct((N, 2 * H), jnp.float32),
    scratch_types=[
        pltpu.VMEM((NCH, CH), jnp.int32),
        pltpu.VMEM((RPT // 5, H), jnp.float32),
        pltpu.VMEM_SHARED((N, H), jnp.float32),
        [pltpu.VMEM((CH, H), jnp.float32)] * NB,
        [pltpu.SemaphoreType.DMA] * NB,
    ],
    compiler_params=_SC_PARAMS,
)
def _sc_scatter(m_h, dst_r, out, idx, stage, acc, buf_m, sem_m):
    cid = lax.axis_index("c")
    sid = lax.axis_index("s")
    r0 = sid * RPT
    e0 = cid * EPC + sid * EPT

    pltpu.sync_copy(dst_r.at[cid, sid], idx)

    def zrow(r, carry):
        for c in range(H // 16):
            stage[r, pl.ds(c * 16, 16)] = jnp.zeros((16,), jnp.float32)
        return carry

    lax.fori_loop(0, RPT // 5, zrow, 0)
    for k in range(5):
        pltpu.sync_copy(stage, acc.at[pl.ds(r0 + k * (RPT // 5), RPT // 5)])
    plsc.subcore_barrier()

    def issue(b, i):
        pltpu.async_copy(m_h.at[pl.ds(e0 + i * CH, CH)], buf_m[b], sem_m[b])

    for b in range(NB):
        issue(b, b)

    def outer(g, carry):
        for b in range(NB):
            i = g * NB + b
            pltpu.make_async_copy(m_h.at[pl.ds(0, CH)], buf_m[b],
                                  sem_m[b]).wait()
            pltpu.sync_copy(buf_m[b], acc.at[idx.at[i]], add=True)

            @pl.when(i + NB < NCH)
            def _():
                issue(b, i + NB)
        return carry

    lax.fori_loop(0, NCH // NB, outer, 0)
    plsc.subcore_barrier()
    for k in range(5):
        rk = r0 + k * (RPT // 5)
        pltpu.sync_copy(acc.at[pl.ds(rk, RPT // 5)], stage)
        pltpu.sync_copy(stage,
                        out.at[pl.ds(rk, RPT // 5), pl.ds(cid * H, H)])


def _tc_init_body(x, w_in, b_in, wa, wb, h_out, a_out, b_out):
    h = jnp.dot(x[:], w_in[:], preferred_element_type=jnp.float32) + b_in[:]
    h_out[:] = h
    a_out[:] = jnp.dot(h, wa[:], preferred_element_type=jnp.float32)
    b_out[:] = jnp.dot(h, wb[:], preferred_element_type=jnp.float32)


_tc_init = pl.pallas_call(
    _tc_init_body,
    out_shape=[
        jax.ShapeDtypeStruct((N, H), jnp.float32),
        jax.ShapeDtypeStruct((N, HID), jnp.float32),
        jax.ShapeDtypeStruct((N, HID), jnp.float32),
    ],
)

BR = 2000


def _tc_msg_body(zg4, ea4, w1cp, b1p, w2d, b2d, s2d, be2d, m_out):
    u = jnp.maximum(
        zg4[:] + jnp.dot(ea4[:], w1cp[:], preferred_element_type=jnp.float32)
        + b1p[:], 0.0)
    even = jnp.maximum(
        jnp.dot(u[:, :64], w2d[:], preferred_element_type=jnp.float32)
        + b2d[:], 0.0) * s2d[:] + be2d[:]
    odd = jnp.maximum(
        jnp.dot(u[:, 64:], w2d[:], preferred_element_type=jnp.float32)
        + b2d[:], 0.0) * s2d[:] + be2d[:]
    m_out[:] = jnp.stack([even, odd], axis=1).reshape(2 * BR, 128)


def _make_tc_msg(off):
    return pl.pallas_call(
        _tc_msg_body,
        grid=(EH // (4 * BR),),
        in_specs=[
            pl.BlockSpec((BR, 128), lambda i: (i, 0)),
            pl.BlockSpec((BR, 16), lambda i: (i + off, 0)),
            pl.BlockSpec((16, 128), lambda i: (0, 0)),
            pl.BlockSpec((1, 128), lambda i: (0, 0)),
            pl.BlockSpec((64, 128), lambda i: (0, 0)),
            pl.BlockSpec((1, 128), lambda i: (0, 0)),
            pl.BlockSpec((1, 128), lambda i: (0, 0)),
            pl.BlockSpec((1, 128), lambda i: (0, 0)),
        ],
        out_specs=pl.BlockSpec((2 * BR, 128), lambda i: (i, 0)),
        out_shape=jax.ShapeDtypeStruct((EH // 2, 128), jnp.float32),
    )


_tc_msg_a = _make_tc_msg(0)
_tc_msg_b = _make_tc_msg(EH // (4 * BR))


def _tc_upd_mid_body(h, pa, pb, u1a, u1b, c1, u2, c2, s2, be2, wa, wb,
                     h_out, a_out, b_out):
    aggr = pa[:, :H] + pa[:, H:] + pb[:, :H] + pb[:, H:]
    v = jnp.maximum(
        jnp.dot(h[:], u1a[:], preferred_element_type=jnp.float32)
        + jnp.dot(aggr, u1b[:], preferred_element_type=jnp.float32)
        + c1[:], 0.0)
    up = jnp.maximum(
        jnp.dot(v, u2[:], preferred_element_type=jnp.float32) + c2[:], 0.0)
    hn = h[:] + up * s2[:] + be2[:]
    h_out[:] = hn
    a_out[:] = jnp.dot(hn, wa[:], preferred_element_type=jnp.float32)
    b_out[:] = jnp.dot(hn, wb[:], preferred_element_type=jnp.float32)


def _tc_upd_last_body(h, pa, pb, u1a, u1b, c1, u2, c2, s2, be2, h_out):
    aggr = pa[:, :H] + pa[:, H:] + pb[:, :H] + pb[:, H:]
    v = jnp.maximum(
        jnp.dot(h[:], u1a[:], preferred_element_type=jnp.float32)
        + jnp.dot(aggr, u1b[:], preferred_element_type=jnp.float32)
        + c1[:], 0.0)
    up = jnp.maximum(
        jnp.dot(v, u2[:], preferred_element_type=jnp.float32) + c2[:], 0.0)
    h_out[:] = h[:] + up * s2[:] + be2[:]


_tc_upd_mid = pl.pallas_call(
    _tc_upd_mid_body,
    out_shape=[
        jax.ShapeDtypeStruct((N, H), jnp.float32),
        jax.ShapeDtypeStruct((N, HID), jnp.float32),
        jax.ShapeDtypeStruct((N, HID), jnp.float32),
    ],
)

_tc_upd_last = pl.pallas_call(
    _tc_upd_last_body,
    out_shape=jax.ShapeDtypeStruct((N, H), jnp.float32),
)


def _tc_final_body(h, batch, w_out, b_out, y_out):
    hv = h[:]
    mu = jnp.mean(hv, axis=-1, keepdims=True)
    var = jnp.mean(hv * hv, axis=-1, keepdims=True) - mu * mu
    hn = (hv - mu) * lax.rsqrt(var + 1e-5)
    gid = lax.broadcasted_iota(jnp.int32, (N, G), 1)
    onehot = jnp.where(batch[:] == gid, 1.0, 0.0).astype(jnp.float32)
    sums = lax.dot_general(onehot, hn, (((0,), (0,)), ((), ())),
                           preferred_element_type=jnp.float32)
    cnt = jnp.sum(onehot, axis=0).reshape(G, 1)
    pooled = sums / jnp.maximum(cnt, 1.0)
    y_out[:] = (jnp.dot(pooled, w_out[:], preferred_element_type=jnp.float32)
                + b_out[:])


_tc_final = pl.pallas_call(
    _tc_final_body,
    out_shape=jax.ShapeDtypeStruct((G, 3), jnp.float32),
)


def _fold_seq(p):
    s = 1.0 / jnp.sqrt(jnp.float32(1.0 + 1e-5))
    g1s = p["g1"] * s
    w1 = p["W1"] * g1s
    b1 = p["b1"] * g1s + p["be1"]
    s2 = p["g2"] * s
    return w1, b1, p["W2"], p["b2"], s2, p["be2"]


def kernel(x, edge_index, edge_attr, batch, params):
    src = edge_index[0]
    dst = edge_index[1]
    halves = []
    for hf in range(2):
        sl = slice(hf * EH, (hf + 1) * EH)
        halves.append(dict(
            src_g=src[sl].reshape(NW, NCH, CH),
            dst_g=dst[sl].reshape(NW, NCH, CH),
            dst_s=dst[sl].reshape(NC, NS, NCH, CH),
        ))
    ea4 = edge_attr.reshape(E // 4, 4 * D_EDGE)
    batch2d = batch.reshape(N, 1)
    bd = jax.scipy.linalg.block_diag

    folded = []
    for lp in params["layers"]:
        w1, b1, w2, b2, s2, be2 = _fold_seq(lp["msg"])
        uw1, uc1, uu2, uc2, us2, ube2 = _fold_seq(lp["upd"])
        w1c = w1[2 * H:]
        folded.append(dict(
            wa=w1[:H], wb=w1[H:2 * H],
            w1cp=bd(w1c, w1c, w1c, w1c),
            b1p=jnp.tile(b1, 4).reshape(1, 128),
            w2d=bd(w2, w2),
            b2d=jnp.tile(b2, 2).reshape(1, 128),
            s2d=jnp.tile(s2, 2).reshape(1, 128),
            be2d=jnp.tile(be2, 2).reshape(1, 128),
            u1a=uw1[:H], u1b=uw1[H:], c1=uc1.reshape(1, HID),
            u2=uu2, c2=uc2.reshape(1, H), us2=us2.reshape(1, H),
            ube2=ube2.reshape(1, H),
        ))

    h, a_tab, b_tab = _tc_init(x, params["Win"], params["bin"].reshape(1, H),
                               folded[0]["wa"], folded[0]["wb"])

    for li in range(L):
        f = folded[li]
        ha, hb = halves
        zg_a = _sc_gather(a_tab, b_tab, ha["dst_g"], ha["src_g"])
        m_a = _tc_msg_a(zg_a, ea4, f["w1cp"], f["b1p"], f["w2d"], f["b2d"],
                        f["s2d"], f["be2d"])
        zg_b = _sc_gather(a_tab, b_tab, hb["dst_g"], hb["src_g"])
        m_b = _tc_msg_b(zg_b, ea4, f["w1cp"], f["b1p"], f["w2d"], f["b2d"],
                        f["s2d"], f["be2d"])
        p_a = _sc_scatter(m_a.reshape(EH, H), ha["dst_s"])
        p_b = _sc_scatter(m_b.reshape(EH, H), hb["dst_s"])
        if li + 1 < L:
            nf = folded[li + 1]
            h, a_tab, b_tab = _tc_upd_mid(
                h, p_a, p_b, f["u1a"], f["u1b"], f["c1"], f["u2"], f["c2"],
                f["us2"], f["ube2"], nf["wa"], nf["wb"])
        else:
            h = _tc_upd_last(h, p_a, p_b, f["u1a"], f["u1b"], f["c1"],
                             f["u2"], f["c2"], f["us2"], f["ube2"])

    return _tc_final(h, batch2d, params["Wout"], params["bout"].reshape(1, 3))
